# baseline (speedup 1.0000x reference)
import functools
from math import sqrt

import jax
import jax.numpy as jnp
from jax.experimental import pallas as pl


K_TOP = 24
NEG_INF = -1e30
BIG_IDX = 2**30


def _cross_ranker_kernel(q_ref, k_ref, out_ref, avg_ref):
    scale = 1.0 / sqrt(q_ref.shape[-1])
    q = q_ref[0]
    k = k_ref[0]

    scores = jax.lax.dot_general(
        q, k, (((1,), (1,)), ((), ())),
        preferred_element_type=jnp.float32)
    scores = scores * scale
    m = jnp.max(scores, axis=-1, keepdims=True)
    e = jnp.exp(scores - m)
    denom = jnp.sum(e, axis=-1, keepdims=True)
    w = (1.0 / 8.0) / denom
    avg = jnp.sum(e * w, axis=0, keepdims=True)
    avg_ref[0] = avg

    v = avg.reshape(64, 128)
    iota = (jax.lax.broadcasted_iota(jnp.int32, (64, 128), 0) * 128
            + jax.lax.broadcasted_iota(jnp.int32, (64, 128), 1))
    top_vals = []
    top_idxs = []
    for _ in range(K_TOP):
        mv = jnp.max(v)
        cand = jnp.where(v == mv, iota, BIG_IDX)
        idx = jnp.min(cand)
        top_vals.append(mv)
        top_idxs.append(idx)
        v = jnp.where(iota == idx, NEG_INF, v)

    mx = top_vals[0]
    exps = [jnp.exp(t - mx) for t in top_vals]
    inv = 1.0 / functools.reduce(lambda a, b: a + b, exps)

    for j in range(K_TOP):
        row = k_ref[0, pl.ds(top_idxs[j], 1), :]
        out_ref[0, pl.ds(j, 1), :] = row * (exps[j] * inv)


def kernel(queries, keys):
    B, L, D = queries.shape
    S = keys.shape[1]
    out, avg = pl.pallas_call(
        _cross_ranker_kernel,
        grid=(B,),
        in_specs=[
            pl.BlockSpec((1, L, D), lambda b: (b, 0, 0)),
            pl.BlockSpec((1, S, D), lambda b: (b, 0, 0)),
        ],
        out_specs=[
            pl.BlockSpec((1, K_TOP, D), lambda b: (b, 0, 0)),
            pl.BlockSpec((1, 1, S), lambda b: (b, 0, 0)),
        ],
        out_shape=[
            jax.ShapeDtypeStruct((B, K_TOP, D), jnp.float32),
            jax.ShapeDtypeStruct((B, 1, S), jnp.float32),
        ],
    )(queries, keys)
    return (out, avg.reshape(B, S))


# speedup vs baseline: 3.4961x; 3.4961x over previous
from math import sqrt

import jax
import jax.numpy as jnp
from jax.experimental import pallas as pl


K_TOP = 24
NEG_INF = -1e30
BIG_IDX = 2**30
BB = 4


def _cross_ranker_kernel(q_ref, k_ref, out_ref, avg_ref):
    scale = 1.0 / sqrt(q_ref.shape[-1])

    avgs = []
    for bb in range(BB):
        scores = jax.lax.dot_general(
            q_ref[bb], k_ref[bb], (((1,), (1,)), ((), ())),
            preferred_element_type=jnp.float32)
        scores = scores * scale
        m = jnp.max(scores, axis=-1, keepdims=True)
        e = jnp.exp(scores - m)
        denom = jnp.sum(e, axis=-1, keepdims=True)
        w = (1.0 / 8.0) / denom
        avg = jnp.sum(e * w, axis=0, keepdims=True)
        avg_ref[bb] = avg
        avgs.append(avg.reshape(1, 64, 128))

    v = jnp.concatenate(avgs, axis=0)
    iota = (jax.lax.broadcasted_iota(jnp.int32, (BB, 64, 128), 1) * 128
            + jax.lax.broadcasted_iota(jnp.int32, (BB, 64, 128), 2))
    top_vals = []
    top_idxs = []
    for _ in range(K_TOP):
        mv = jnp.max(v, axis=(1, 2), keepdims=True)
        cand = jnp.where(v == mv, iota, BIG_IDX)
        idx = jnp.min(cand, axis=(1, 2), keepdims=True)
        top_vals.append(mv)
        top_idxs.append(idx)
        v = jnp.where(iota == idx, NEG_INF, v)

    tv = jnp.concatenate(top_vals, axis=2)
    ex = jnp.exp(tv - top_vals[0])
    wts = ex / jnp.sum(ex, axis=2, keepdims=True)

    for bb in range(BB):
        for j in range(K_TOP):
            row = k_ref[bb, pl.ds(top_idxs[j][bb, 0, 0], 1), :]
            out_ref[bb, pl.ds(j, 1), :] = row * wts[bb, :, j:j + 1]


def kernel(queries, keys):
    B, L, D = queries.shape
    S = keys.shape[1]
    out, avg = pl.pallas_call(
        _cross_ranker_kernel,
        grid=(B // BB,),
        in_specs=[
            pl.BlockSpec((BB, L, D), lambda b: (b, 0, 0)),
            pl.BlockSpec((BB, S, D), lambda b: (b, 0, 0)),
        ],
        out_specs=[
            pl.BlockSpec((BB, K_TOP, D), lambda b: (b, 0, 0)),
            pl.BlockSpec((BB, 1, S), lambda b: (b, 0, 0)),
        ],
        out_shape=[
            jax.ShapeDtypeStruct((B, K_TOP, D), jnp.float32),
            jax.ShapeDtypeStruct((B, 1, S), jnp.float32),
        ],
    )(queries, keys)
    return (out, avg.reshape(B, S))
